# baseline (speedup 1.0000x reference)
import functools
import math

import jax
import jax.numpy as jnp
from jax import lax
from jax.experimental import pallas as pl
from jax.experimental.pallas import tpu as pltpu
from jax.experimental.pallas import tpu_sc as plsc

NUM_CORES = 2
NUM_SUBCORES = 16
NW = NUM_CORES * NUM_SUBCORES
LANES = 16
CHUNK = 16


def _make_kernel(B, S, D, V):
    s_per_w = S // NW
    n_groups = s_per_w // CHUNK
    rows_g = B * CHUNK
    scale = jnp.float32(math.sqrt(float(D)))
    d_regs = D // LANES

    mesh = plsc.VectorSubcoreMesh(
        core_axis_name="c", subcore_axis_name="s",
        num_cores=NUM_CORES, num_subcores=NUM_SUBCORES)

    @functools.partial(
        pl.kernel,
        mesh=mesh,
        out_type=jax.ShapeDtypeStruct((B * S, D), jnp.float32),
        scratch_types=[
            pltpu.VMEM((n_groups, rows_g), jnp.int32),
            pltpu.VMEM((n_groups, rows_g), jnp.int32),
            pltpu.VMEM((rows_g, D), jnp.float32),
            pltpu.VMEM((rows_g, D), jnp.float32),
            pltpu.VMEM((CHUNK, D), jnp.float32),
            pltpu.VMEM((CHUNK, D), jnp.float32),
            pltpu.SemaphoreType.DMA,
            pltpu.SemaphoreType.DMA,
            pltpu.SemaphoreType.DMA,
        ],
    )
    def emb_kernel(xcat_hbm, rows_hbm, wt_hbm, pos_hbm, out_hbm,
                   idx_v, row_v, wbuf0, wbuf1, pbuf0, pbuf1,
                   gsem, ssem, psem):
        wid = lax.axis_index("s") * NUM_CORES + lax.axis_index("c")
        s_base = wid * s_per_w
        pltpu.sync_copy(xcat_hbm.at[wid], idx_v)
        pltpu.sync_copy(rows_hbm.at[wid], row_v)

        wbufs = (wbuf0, wbuf1)
        pbufs = (pbuf0, pbuf1)

        def start_gather(g, buf):
            pltpu.async_copy(wt_hbm.at[idx_v.at[g]], buf, gsem)

        def wait_gather(buf):
            pltpu.make_async_copy(wt_hbm.at[pl.ds(0, rows_g)], buf, gsem).wait()

        def start_pos(g, buf):
            pltpu.async_copy(
                pos_hbm.at[pl.ds(s_base + g * CHUNK, CHUNK)], buf, psem)

        def wait_pos(buf):
            pltpu.make_async_copy(
                pos_hbm.at[pl.ds(0, CHUNK)], buf, psem).wait()

        def start_store(g, buf):
            pltpu.async_copy(buf, out_hbm.at[row_v.at[g]], ssem)

        def wait_store(buf):
            pltpu.make_async_copy(buf, out_hbm.at[pl.ds(0, rows_g)], ssem).wait()

        start_gather(0, wbuf0)
        start_pos(0, pbuf0)

        def pair_body(gp, _):
            for q in range(2):
                g = gp * 2 + q
                wb = wbufs[q]
                wb_other = wbufs[1 - q]
                pb = pbufs[q]
                pb_other = pbufs[1 - q]

                @pl.when(g < n_groups - 1)
                def _():
                    @pl.when(g >= 1)
                    def _():
                        wait_store(wb_other)
                    start_gather(g + 1, wb_other)
                    start_pos(g + 1, pb_other)

                wait_gather(wb)
                wait_pos(pb)

                def row_body(r, _):
                    for j in range(d_regs):
                        sl = pl.ds(j * LANES, LANES)
                        p = pb[r, sl]
                        for b in range(B):
                            wb[b * CHUNK + r, sl] = (wb[b * CHUNK + r, sl] + p) * scale
                    return 0

                lax.fori_loop(0, CHUNK, row_body, 0)
                start_store(g, wb)
            return 0

        lax.fori_loop(0, n_groups // 2, pair_body, 0)
        wait_store(wbuf0)
        wait_store(wbuf1)

    return emb_kernel


def kernel(x, word_table, pos_table):
    B, S = x.shape
    V, D = word_table.shape
    s_per_w = S // NW
    n_groups = s_per_w // CHUNK
    xcat = (x.reshape(B, NW, n_groups, CHUNK)
             .transpose(1, 2, 0, 3)
             .reshape(NW, n_groups, B * CHUNK))
    b_ids = jnp.arange(B, dtype=jnp.int32)
    s_ids = jnp.arange(S, dtype=jnp.int32)
    rows = (b_ids[:, None] * S + s_ids[None, :])
    rows = (rows.reshape(B, NW, n_groups, CHUNK)
                .transpose(1, 2, 0, 3)
                .reshape(NW, n_groups, B * CHUNK))
    emb = _make_kernel(B, S, D, V)
    out = emb(xcat, rows, word_table, pos_table[:S])
    return out.reshape(B, S, D)


# speedup vs baseline: 1.8602x; 1.8602x over previous
import functools
import math

import jax
import jax.numpy as jnp
from jax import lax
from jax.experimental import pallas as pl
from jax.experimental.pallas import tpu as pltpu
from jax.experimental.pallas import tpu_sc as plsc

NUM_CORES = 2
NUM_SUBCORES = 16
NW = NUM_CORES * NUM_SUBCORES
LANES = 16
CHUNK = 32


def _make_kernel(B, S, D, V):
    s_per_w = S // NW
    n_chunks = s_per_w // CHUNK
    n_units = n_chunks * B
    scale = jnp.float32(math.sqrt(float(D)))
    d_regs = D // LANES

    mesh = plsc.VectorSubcoreMesh(
        core_axis_name="c", subcore_axis_name="s",
        num_cores=NUM_CORES, num_subcores=NUM_SUBCORES)

    @functools.partial(
        pl.kernel,
        mesh=mesh,
        out_type=jax.ShapeDtypeStruct((B * S, D), jnp.float32),
        scratch_types=[
            pltpu.VMEM((B, s_per_w), jnp.int32),
            pltpu.VMEM((CHUNK, D), jnp.float32),
            pltpu.VMEM((CHUNK, D), jnp.float32),
            pltpu.VMEM((CHUNK, D), jnp.float32),
            pltpu.VMEM((CHUNK, D), jnp.float32),
            pltpu.SemaphoreType.DMA,
            pltpu.SemaphoreType.DMA,
            pltpu.SemaphoreType.DMA,
        ],
    )
    def emb_kernel(x_hbm, wt_hbm, pos_hbm, out_hbm,
                   idx_v, wbuf0, wbuf1, pbuf0, pbuf1, gsem, ssem, psem):
        wid = lax.axis_index("s") * NUM_CORES + lax.axis_index("c")
        s_base = wid * s_per_w
        for b in range(B):
            pltpu.sync_copy(x_hbm.at[b, pl.ds(s_base, s_per_w)],
                            idx_v.at[b])

        wbufs = (wbuf0, wbuf1)
        pbufs = (pbuf0, pbuf1)

        def start_gather(u, buf):
            bb = lax.rem(u, B)
            ci = u // B
            pltpu.async_copy(
                wt_hbm.at[idx_v.at[bb, pl.ds(ci * CHUNK, CHUNK)]], buf, gsem)

        def wait_gather(buf):
            pltpu.make_async_copy(wt_hbm.at[pl.ds(0, CHUNK)], buf, gsem).wait()

        def start_pos(ci, buf):
            pltpu.async_copy(
                pos_hbm.at[pl.ds(s_base + ci * CHUNK, CHUNK)], buf, psem)

        def wait_pos(buf):
            pltpu.make_async_copy(
                pos_hbm.at[pl.ds(0, CHUNK)], buf, psem).wait()

        def start_store(u, buf):
            bb = lax.rem(u, B)
            ci = u // B
            row = bb * S + s_base + ci * CHUNK
            pltpu.async_copy(buf, out_hbm.at[pl.ds(row, CHUNK)], ssem)

        def wait_store(buf):
            pltpu.make_async_copy(buf, out_hbm.at[pl.ds(0, CHUNK)], ssem).wait()

        start_gather(0, wbuf0)
        start_pos(0, pbuf0)

        def oct_body(up, _):
            for uu in range(8):
                u = up * 8 + uu
                wb = wbufs[uu % 2]
                wb_other = wbufs[1 - uu % 2]
                pb = pbufs[(uu // 4) % 2]
                pb_other = pbufs[1 - (uu // 4) % 2]

                @pl.when(u < n_units - 1)
                def _():
                    @pl.when(u >= 1)
                    def _():
                        wait_store(wb_other)
                    start_gather(u + 1, wb_other)

                if uu % 4 == 0:
                    ci = up * 2 + uu // 4

                    @pl.when(ci < n_chunks - 1)
                    def _():
                        start_pos(ci + 1, pb_other)

                wait_gather(wb)
                if uu % 4 == 0:
                    wait_pos(pb)

                def row_body(r, _):
                    for j in range(d_regs):
                        sl = pl.ds(j * LANES, LANES)
                        wb[r, sl] = (wb[r, sl] + pb[r, sl]) * scale
                    return 0

                lax.fori_loop(0, CHUNK, row_body, 0)
                start_store(u, wb)
            return 0

        lax.fori_loop(0, n_units // 8, oct_body, 0)
        wait_store(wbuf0)
        wait_store(wbuf1)

    return emb_kernel


def kernel(x, word_table, pos_table):
    B, S = x.shape
    V, D = word_table.shape
    emb = _make_kernel(B, S, D, V)
    out = emb(x, word_table, pos_table[:S])
    return out.reshape(B, S, D)
